# baseline (speedup 1.0000x reference)
import functools

import jax
import jax.numpy as jnp
from jax import lax
from jax.experimental import pallas as pl
from jax.experimental.pallas import tpu as pltpu
from jax.experimental.pallas import tpu_sc as plsc

UPDATE_N = 100000
VOCAB_N = 1000000
D = 32
L = 16
NC, NS = 2, 16
NW = NC * NS
B_ROWS = 4096 * 200
ROWS_PER_W = B_ROWS // NW
CHUNK = 1024
N_CHUNKS = ROWS_PER_W // CHUNK
SUB = 8

_mesh = plsc.VectorSubcoreMesh(core_axis_name="c", subcore_axis_name="s")


@functools.partial(
    pl.kernel,
    out_type=jax.ShapeDtypeStruct((B_ROWS, D), jnp.float32),
    mesh=_mesh,
    compiler_params=pltpu.CompilerParams(use_tc_tiling_on_sc=False),
    scratch_types=[
        pltpu.VMEM((CHUNK,), jnp.int32),
        pltpu.VMEM((CHUNK,), jnp.int32),
        pltpu.VMEM((CHUNK, D), jnp.float32),
        pltpu.VMEM((CHUNK, D), jnp.float32),
        pltpu.SemaphoreType.DMA,
        pltpu.SemaphoreType.DMA,
    ],
)
def _sc_lookup(idx_hbm, wu_hbm, wf_hbm, out_hbm, idxv, uidxv, ubuf, fbuf,
               sem_u, sem_f):
    wid = lax.axis_index("s") * NC + lax.axis_index("c")
    base = wid * ROWS_PER_W

    def chunk_body(ci, carry):
        start = base + ci * CHUNK
        pltpu.sync_copy(idx_hbm.at[pl.ds(start, CHUNK)], idxv)

        def clamp_body(j, carry2):
            v = idxv[pl.ds(j * L, L)]
            uidxv[pl.ds(j * L, L)] = jnp.minimum(v, UPDATE_N - 1)
            return carry2

        lax.fori_loop(0, CHUNK // L, clamp_body, 0)

        copies = []
        for s in range(SUB):
            sl = pl.ds(s * (CHUNK // SUB), CHUNK // SUB)
            copies.append(
                pltpu.async_copy(wu_hbm.at[uidxv.at[sl]], ubuf.at[sl], sem_u))
            copies.append(
                pltpu.async_copy(wf_hbm.at[idxv.at[sl]], fbuf.at[sl], sem_f))
        for c in copies:
            c.wait()

        def blend_body(g, carry2):
            vi = idxv[pl.ds(g * L, L)]
            for k in range(L):
                r = g * L + k
                m = vi[k] < UPDATE_N
                for h in range(D // L):
                    u = ubuf[r, pl.ds(h * L, L)]
                    f = fbuf[r, pl.ds(h * L, L)]
                    fbuf[r, pl.ds(h * L, L)] = jnp.where(m, u, f)
            return carry2

        lax.fori_loop(0, CHUNK // L, blend_body, 0)

        pltpu.sync_copy(fbuf, out_hbm.at[pl.ds(start, CHUNK)])
        return carry

    lax.fori_loop(0, N_CHUNKS, chunk_body, 0)


def kernel(inp, W_update, W_fixed):
    idx = inp.reshape(B_ROWS).astype(jnp.int32)
    out = _sc_lookup(idx, W_update, W_fixed)
    return out.reshape(inp.shape[0], inp.shape[1], D)


# speedup vs baseline: 3.7869x; 3.7869x over previous
import functools

import jax
import jax.numpy as jnp
from jax import lax
from jax.experimental import pallas as pl
from jax.experimental.pallas import tpu as pltpu
from jax.experimental.pallas import tpu_sc as plsc

UPDATE_N = 100000
VOCAB_N = 1000000
D = 32
L = 16
NC, NS = 2, 16
NW = NC * NS
B_ROWS = 4096 * 200
ROWS_PER_W = B_ROWS // NW
CHUNK = 1024
N_CHUNKS = ROWS_PER_W // CHUNK
TOT_CHUNKS = NW * N_CHUNKS
T_WORDS = (UPDATE_N + VOCAB_N) * D

_mesh = plsc.VectorSubcoreMesh(core_axis_name="c", subcore_axis_name="s")


@functools.partial(
    pl.kernel,
    out_type=jax.ShapeDtypeStruct((TOT_CHUNKS, D, CHUNK), jnp.float32),
    mesh=_mesh,
    compiler_params=pltpu.CompilerParams(use_tc_tiling_on_sc=False),
    scratch_types=[
        pltpu.VMEM((CHUNK,), jnp.int32),
        pltpu.VMEM((D, CHUNK), jnp.float32),
        pltpu.SemaphoreType.DMA,
    ],
)
def _sc_lookup(idx_hbm, tab_hbm, out_hbm, idxv, bufT, sem):
    wid = lax.axis_index("s") * NC + lax.axis_index("c")
    base = wid * ROWS_PER_W

    def chunk_body(ci, carry):
        start = base + ci * CHUNK
        pltpu.sync_copy(idx_hbm.at[pl.ds(start, CHUNK)], idxv)

        def group_body(g, carry2):
            v = idxv[pl.ds(g * L, L)]
            addr = v * D + jnp.where(v < UPDATE_N, 0, UPDATE_N * D)
            for d in range(D):
                pltpu.async_copy(tab_hbm.at[addr + d],
                                 bufT.at[d, pl.ds(g * L, L)], sem)
            return carry2

        lax.fori_loop(0, CHUNK // L, group_body, 0)
        pltpu.make_async_copy(out_hbm.at[0], bufT, sem).wait()
        pltpu.sync_copy(bufT, out_hbm.at[wid * N_CHUNKS + ci])
        return carry

    lax.fori_loop(0, N_CHUNKS, chunk_body, 0)


def kernel(inp, W_update, W_fixed):
    idx = inp.reshape(B_ROWS).astype(jnp.int32)
    tab = jnp.concatenate(
        [W_update.reshape(UPDATE_N * D), W_fixed.reshape(VOCAB_N * D)])
    out3 = _sc_lookup(idx, tab)
    out = out3.transpose(0, 2, 1).reshape(B_ROWS, D)
    return out.reshape(inp.shape[0], inp.shape[1], D)


# speedup vs baseline: 7.4318x; 1.9625x over previous
import functools

import jax
import jax.numpy as jnp
from jax import lax
from jax.experimental import pallas as pl
from jax.experimental.pallas import tpu as pltpu
from jax.experimental.pallas import tpu_sc as plsc

UPDATE_N = 100000
VOCAB_N = 1000000
D = 32
L = 16
NC, NS = 2, 16
NW = NC * NS
B_ROWS = 4096 * 200
ROWS_PER_W = B_ROWS // NW
CHUNK = 1024
N_CHUNKS = ROWS_PER_W // CHUNK
SUB = 8

_mesh = plsc.VectorSubcoreMesh(core_axis_name="c", subcore_axis_name="s")


@functools.partial(
    pl.kernel,
    out_type=jax.ShapeDtypeStruct((B_ROWS, D), jnp.float32),
    mesh=_mesh,
    compiler_params=pltpu.CompilerParams(use_tc_tiling_on_sc=False),
    scratch_types=[
        pltpu.VMEM((CHUNK,), jnp.int32),
        pltpu.VMEM((CHUNK,), jnp.int32),
        pltpu.VMEM((CHUNK, D), jnp.float32),
        pltpu.SemaphoreType.DMA,
    ],
)
def _sc_lookup(idx_hbm, tab_hbm, out_hbm, idxv, ridxv, buf, sem):
    wid = lax.axis_index("s") * NC + lax.axis_index("c")
    base = wid * ROWS_PER_W

    def chunk_body(ci, carry):
        start = base + ci * CHUNK
        pltpu.sync_copy(idx_hbm.at[pl.ds(start, CHUNK)], idxv)

        def remap_body(j, carry2):
            v = idxv[pl.ds(j * L, L)]
            ridxv[pl.ds(j * L, L)] = v + jnp.where(v < UPDATE_N, 0, UPDATE_N)
            return carry2

        lax.fori_loop(0, CHUNK // L, remap_body, 0)

        copies = []
        for s in range(SUB):
            sl = pl.ds(s * (CHUNK // SUB), CHUNK // SUB)
            copies.append(
                pltpu.async_copy(tab_hbm.at[ridxv.at[sl]], buf.at[sl], sem))
        for c in copies:
            c.wait()

        pltpu.sync_copy(buf, out_hbm.at[pl.ds(start, CHUNK)])
        return carry

    lax.fori_loop(0, N_CHUNKS, chunk_body, 0)


def kernel(inp, W_update, W_fixed):
    idx = inp.reshape(B_ROWS).astype(jnp.int32)
    tab = jnp.concatenate([W_update, W_fixed])
    out = _sc_lookup(idx, tab)
    return out.reshape(inp.shape[0], inp.shape[1], D)


# speedup vs baseline: 7.5754x; 1.0193x over previous
import functools

import jax
import jax.numpy as jnp
from jax import lax
from jax.experimental import pallas as pl
from jax.experimental.pallas import tpu as pltpu
from jax.experimental.pallas import tpu_sc as plsc

UPDATE_N = 100000
VOCAB_N = 1000000
D = 32
L = 16
NC, NS = 2, 16
NW = NC * NS
B_ROWS = 4096 * 200
ROWS_PER_W = B_ROWS // NW
CHUNK = 1024
N_CHUNKS = ROWS_PER_W // CHUNK
SUB = 8

_mesh = plsc.VectorSubcoreMesh(core_axis_name="c", subcore_axis_name="s")


@functools.partial(
    pl.kernel,
    out_type=jax.ShapeDtypeStruct((B_ROWS, D), jnp.float32),
    mesh=_mesh,
    compiler_params=pltpu.CompilerParams(use_tc_tiling_on_sc=False),
    scratch_types=[
        pltpu.VMEM((ROWS_PER_W,), jnp.int32),
        pltpu.VMEM((CHUNK, D), jnp.float32),
        pltpu.VMEM((CHUNK, D), jnp.float32),
        pltpu.SemaphoreType.DMA,
        pltpu.SemaphoreType.DMA,
        pltpu.SemaphoreType.DMA,
        pltpu.SemaphoreType.DMA,
    ],
)
def _sc_lookup(idx_hbm, tab_hbm, out_hbm, idxv, buf_a, buf_b,
               sg_a, sg_b, sw_a, sw_b):
    wid = lax.axis_index("s") * NC + lax.axis_index("c")
    base = wid * ROWS_PER_W

    pltpu.sync_copy(idx_hbm.at[pl.ds(base, ROWS_PER_W)], idxv)

    def remap_body(j, carry):
        v = idxv[pl.ds(j * L, L)]
        idxv[pl.ds(j * L, L)] = v + jnp.where(v < UPDATE_N, 0, UPDATE_N)
        return carry

    lax.fori_loop(0, ROWS_PER_W // L, remap_body, 0)

    bufs = (buf_a, buf_b)
    sgs = (sg_a, sg_b)
    sws = (sw_a, sw_b)

    def issue_gather(ci):
        b = ci % 2
        copies = []
        for s in range(SUB):
            sl = pl.ds(ci * CHUNK + s * (CHUNK // SUB), CHUNK // SUB)
            dl = pl.ds(s * (CHUNK // SUB), CHUNK // SUB)
            copies.append(pltpu.async_copy(
                tab_hbm.at[idxv.at[sl]], bufs[b].at[dl], sgs[b]))
        return copies

    gathers = [None, None]
    writes = [None, None]
    gathers[0] = issue_gather(0)

    for ci in range(N_CHUNKS):
        b = ci % 2
        for c in gathers[b]:
            c.wait()
        if ci + 1 < N_CHUNKS:
            nb = (ci + 1) % 2
            if writes[nb] is not None:
                writes[nb].wait()
            gathers[nb] = issue_gather(ci + 1)
        writes[b] = pltpu.async_copy(
            bufs[b], out_hbm.at[pl.ds(base + ci * CHUNK, CHUNK)], sws[b])

    for w in writes:
        if w is not None:
            w.wait()


def kernel(inp, W_update, W_fixed):
    idx = inp.reshape(B_ROWS).astype(jnp.int32)
    tab = jnp.concatenate([W_update, W_fixed])
    out = _sc_lookup(idx, tab)
    return out.reshape(inp.shape[0], inp.shape[1], D)
